# baseline (speedup 1.0000x reference)
import functools

import jax
import jax.numpy as jnp
from jax import lax
from jax.experimental import pallas as pl
from jax.experimental.pallas import tpu as pltpu
from jax.experimental.pallas import tpu_sc as plsc

VOCAB = 8192
NTOK = 8192
BLK = 128
NBLK = NTOK // BLK

NC = 2
NS = 16
NW = NC * NS
PERW = NTOK // NW
ROWW = 16
NROW = VOCAB * VOCAB // ROWW


def _gather_lse_body(idx_ref, emb_hbm, out_hbm, lsesum_ref, buf, sem, osem, acc):
    i = pl.program_id(0)

    def start_fetch(block, slot):
        base = block * BLK
        for j in range(BLK):
            row = idx_ref[base + j]
            pltpu.make_async_copy(
                emb_hbm.at[pl.ds(row, 1)],
                buf.at[slot, pl.ds(j, 1)],
                sem.at[slot],
            ).start()

    def out_dma(block, slot):
        return pltpu.make_async_copy(
            buf.at[slot], out_hbm.at[pl.ds(block * BLK, BLK)], osem.at[slot]
        )

    @pl.when(i == 0)
    def _():
        acc[0] = 0.0
        start_fetch(0, 0)

    @pl.when(i + 1 < NBLK)
    def _():
        nslot = (i + 1) % 2
        @pl.when(i >= 1)
        def _():
            out_dma(i - 1, nslot).wait()
        start_fetch(i + 1, nslot)

    slot = i % 2
    pltpu.make_async_copy(
        emb_hbm.at[pl.ds(0, BLK)], buf.at[slot], sem.at[slot]
    ).wait()

    x = buf[slot]
    m = jnp.max(x, axis=1)
    s = jnp.sum(jnp.exp(x - m[:, None]), axis=1)
    acc[0] += jnp.sum(m + jnp.log(s))
    lsesum_ref[0] = acc[0]

    out_dma(i, slot).start()
    @pl.when(i == NBLK - 1)
    def _():
        out_dma(i, slot).wait()
        out_dma(i - 1, (i - 1) % 2).wait()


def _gather_lse(idx_flat, emb):
    grid_spec = pltpu.PrefetchScalarGridSpec(
        num_scalar_prefetch=1,
        grid=(NBLK,),
        in_specs=[pl.BlockSpec(memory_space=pl.ANY)],
        out_specs=[
            pl.BlockSpec(memory_space=pl.ANY),
            pl.BlockSpec(memory_space=pltpu.SMEM),
        ],
        scratch_shapes=[
            pltpu.VMEM((2, BLK, VOCAB), jnp.float32),
            pltpu.SemaphoreType.DMA((2,)),
            pltpu.SemaphoreType.DMA((2,)),
            pltpu.SMEM((1,), jnp.float32),
        ],
    )
    return pl.pallas_call(
        _gather_lse_body,
        grid_spec=grid_spec,
        out_shape=[
            jax.ShapeDtypeStruct((NTOK, VOCAB), jnp.float32),
            jax.ShapeDtypeStruct((1,), jnp.float32),
        ],
    )(idx_flat, emb)


def _sc_pick_body(embf_hbm, idx_hbm, tgt_hbm, out_hbm,
                  idx_v, tgt_v, fid_v, got_v, acc_v, sem):
    wid = lax.axis_index("s") * NC + lax.axis_index("c")
    base = wid * PERW
    pltpu.sync_copy(idx_hbm.at[pl.ds(base, PERW)], idx_v)
    pltpu.sync_copy(tgt_hbm.at[pl.ds(base, PERW)], tgt_v)
    for c in range(PERW // 16):
        sl = pl.ds(c * 16, 16)
        i16 = idx_v[sl]
        t16 = tgt_v[sl]
        fid_v[sl] = (
            lax.shift_left(lax.shift_right_logical(i16, 3), 16)
            + lax.shift_left(lax.shift_right_logical(t16, 7), 10)
            + lax.shift_left(jnp.bitwise_and(i16, 7), 7)
            + jnp.bitwise_and(t16, 127)
        )
    pltpu.async_copy(embf_hbm.at[fid_v], got_v, sem).wait()
    acc = jnp.zeros((16,), jnp.float32)
    for c in range(PERW // 16):
        acc = acc + got_v[pl.ds(c * 16, 16)]
    acc_v[...] = acc
    pltpu.sync_copy(acc_v, out_hbm.at[wid])


def _sc_pick(embf, idx_flat, tgt_flat):
    mesh = plsc.VectorSubcoreMesh(core_axis_name="c", subcore_axis_name="s")
    run = functools.partial(
        pl.kernel,
        out_type=jax.ShapeDtypeStruct((NW, 16), jnp.float32),
        mesh=mesh,
        scratch_types=[
            pltpu.VMEM((PERW,), jnp.int32),
            pltpu.VMEM((PERW,), jnp.int32),
            pltpu.VMEM((PERW,), jnp.int32),
            pltpu.VMEM((PERW,), jnp.float32),
            pltpu.VMEM((16,), jnp.float32),
            pltpu.SemaphoreType.DMA,
        ],
    )(_sc_pick_body)
    return run(embf, idx_flat, tgt_flat)


def kernel(idx, targets, emb):
    Bd, Td = idx.shape
    idx_flat = idx.reshape(NTOK).astype(jnp.int32)
    tgt_flat = targets.reshape(NTOK).astype(jnp.int32)
    logits_flat, lsesum = _gather_lse(idx_flat, emb)
    embf_phys = emb.reshape(VOCAB // 8, 8, VOCAB // 128, 128).transpose(
        0, 2, 1, 3).reshape(VOCAB * VOCAB)
    partials = _sc_pick(embf_phys, idx_flat, tgt_flat)
    loss = (lsesum[0] - jnp.sum(partials)) / NTOK
    return logits_flat.reshape(Bd, Td, VOCAB), loss


# speedup vs baseline: 1.1266x; 1.1266x over previous
import functools

import jax
import jax.numpy as jnp
from jax import lax
from jax.experimental import pallas as pl
from jax.experimental.pallas import tpu as pltpu
from jax.experimental.pallas import tpu_sc as plsc

VOCAB = 8192
NTOK = 8192
BLK = 256
NBLK = NTOK // BLK
NSLOT = 6
PREF = 3
NQ = 4

NC = 2
NS = 16
NW = NC * NS
PERW = NTOK // NW


def _gather_lse_body(idx_ref, emb_hbm, out_hbm, lsesum_ref, buf, sem, osem, acc):
    i = pl.program_id(0)

    def start_fetch(block, slot):
        base = block * BLK
        for j in range(BLK):
            row = idx_ref[base + j]
            pltpu.make_async_copy(
                emb_hbm.at[pl.ds(row, 1)],
                buf.at[slot, pl.ds(j, 1)],
                sem.at[slot, j % NQ],
            ).start(priority=j % 2)

    def wait_fetch(slot):
        for q in range(NQ):
            pltpu.make_async_copy(
                emb_hbm.at[pl.ds(0, BLK // NQ)],
                buf.at[slot, pl.ds(0, BLK // NQ)],
                sem.at[slot, q],
            ).wait()

    def out_dma(block, slot):
        return pltpu.make_async_copy(
            buf.at[slot], out_hbm.at[pl.ds(block * BLK, BLK)], osem.at[slot]
        )

    @pl.when(i == 0)
    def _():
        acc[0] = 0.0
        for b in range(PREF):
            start_fetch(b, b)

    @pl.when(i + PREF < NBLK)
    def _():
        nblock = i + PREF
        nslot = (nblock) % NSLOT
        @pl.when(nblock >= NSLOT)
        def _():
            out_dma(nblock - NSLOT, nslot).wait()
        start_fetch(nblock, nslot)

    slot = i % NSLOT
    wait_fetch(slot)
    out_dma(i, slot).start()

    x = buf[slot]
    m = jnp.max(x, axis=1)
    s = jnp.sum(jnp.exp(x - m[:, None]), axis=1)
    acc[0] += jnp.sum(m + jnp.log(s))
    lsesum_ref[0] = acc[0]

    @pl.when(i == NBLK - 1)
    def _():
        for d in range(NSLOT):
            out_dma(i - d, (i - d) % NSLOT).wait()


def _gather_lse(idx_flat, emb):
    grid_spec = pltpu.PrefetchScalarGridSpec(
        num_scalar_prefetch=1,
        grid=(NBLK,),
        in_specs=[pl.BlockSpec(memory_space=pl.ANY)],
        out_specs=[
            pl.BlockSpec(memory_space=pl.ANY),
            pl.BlockSpec(memory_space=pltpu.SMEM),
        ],
        scratch_shapes=[
            pltpu.VMEM((NSLOT, BLK, VOCAB), jnp.float32),
            pltpu.SemaphoreType.DMA((NSLOT, NQ)),
            pltpu.SemaphoreType.DMA((NSLOT,)),
            pltpu.SMEM((1,), jnp.float32),
        ],
    )
    return pl.pallas_call(
        _gather_lse_body,
        grid_spec=grid_spec,
        out_shape=[
            jax.ShapeDtypeStruct((NTOK, VOCAB), jnp.float32),
            jax.ShapeDtypeStruct((1,), jnp.float32),
        ],
    )(idx_flat, emb)


def _sc_pick_body(embf_hbm, idx_hbm, tgt_hbm, out_hbm,
                  idx_v, tgt_v, fid_v, got_v, acc_v, sem):
    wid = lax.axis_index("s") * NC + lax.axis_index("c")
    base = wid * PERW
    pltpu.sync_copy(idx_hbm.at[pl.ds(base, PERW)], idx_v)
    pltpu.sync_copy(tgt_hbm.at[pl.ds(base, PERW)], tgt_v)
    for c in range(PERW // 16):
        sl = pl.ds(c * 16, 16)
        i16 = idx_v[sl]
        t16 = tgt_v[sl]
        fid_v[sl] = (
            lax.shift_left(lax.shift_right_logical(i16, 3), 16)
            + lax.shift_left(lax.shift_right_logical(t16, 7), 10)
            + lax.shift_left(jnp.bitwise_and(i16, 7), 7)
            + jnp.bitwise_and(t16, 127)
        )
    pltpu.async_copy(embf_hbm.at[fid_v], got_v, sem).wait()
    acc = jnp.zeros((16,), jnp.float32)
    for c in range(PERW // 16):
        acc = acc + got_v[pl.ds(c * 16, 16)]
    acc_v[...] = acc
    pltpu.sync_copy(acc_v, out_hbm.at[wid])


def _sc_pick(embf, idx_flat, tgt_flat):
    mesh = plsc.VectorSubcoreMesh(core_axis_name="c", subcore_axis_name="s")
    run = functools.partial(
        pl.kernel,
        out_type=jax.ShapeDtypeStruct((NW, 16), jnp.float32),
        mesh=mesh,
        scratch_types=[
            pltpu.VMEM((PERW,), jnp.int32),
            pltpu.VMEM((PERW,), jnp.int32),
            pltpu.VMEM((PERW,), jnp.int32),
            pltpu.VMEM((PERW,), jnp.float32),
            pltpu.VMEM((16,), jnp.float32),
            pltpu.SemaphoreType.DMA,
        ],
    )(_sc_pick_body)
    return run(embf, idx_flat, tgt_flat)


def kernel(idx, targets, emb):
    Bd, Td = idx.shape
    idx_flat = idx.reshape(NTOK).astype(jnp.int32)
    tgt_flat = targets.reshape(NTOK).astype(jnp.int32)
    logits_flat, lsesum = _gather_lse(idx_flat, emb)
    embf_phys = emb.reshape(VOCAB // 8, 8, VOCAB // 128, 128).transpose(
        0, 2, 1, 3).reshape(VOCAB * VOCAB)
    partials = _sc_pick(embf_phys, idx_flat, tgt_flat)
    loss = (lsesum[0] - jnp.sum(partials)) / NTOK
    return logits_flat.reshape(Bd, Td, VOCAB), loss
